# baseline (speedup 1.0000x reference)
import functools

import jax
import jax.numpy as jnp
from jax import lax
from jax.experimental import pallas as pl
from jax.experimental.pallas import tpu as pltpu
from jax.experimental.pallas import tpu_sc as plsc

BATCH = 16384
EMB = 32
_NC = 2
_NS = 16
_NW = _NC * _NS
_BPW = BATCH // _NW
_CHUNK = 128
_NCHUNK = _BPW // _CHUNK


def _sc_gather_body(item_tab, user_tab, dur_tab, gids_hbm,
                    item_out, user_out, dur_out,
                    idx_v, bi0, bu0, bd0, bi1, bu1, bd1, sem0, sem1):
  wid = lax.axis_index("s") * _NC + lax.axis_index("c")
  base = wid * _BPW
  row0 = wid * _NCHUNK
  pltpu.sync_copy(gids_hbm.at[:, pl.ds(row0, _NCHUNK), :], idx_v)
  bufs = ((bi0, bu0, bd0), (bi1, bu1, bd1))
  sems = (sem0, sem1)

  def fire(c):
    bi, bu, bd = bufs[c % 2]
    s = sems[c % 2]
    return (pltpu.async_copy(item_tab.at[idx_v.at[0, c]], bi, s),
            pltpu.async_copy(user_tab.at[idx_v.at[1, c]], bu, s),
            pltpu.async_copy(dur_tab.at[idx_v.at[2, c]], bd, s))

  def drain_writeback(c, handles):
    for h in handles:
      h.wait()
    bi, bu, bd = bufs[c % 2]
    sl = pl.ds(base + c * _CHUNK, _CHUNK)
    pltpu.sync_copy(bi, item_out.at[sl])
    pltpu.sync_copy(bu, user_out.at[sl])
    pltpu.sync_copy(bd, dur_out.at[sl])

  pending = fire(0)
  for c in range(1, _NCHUNK):
    nxt = fire(c)
    drain_writeback(c - 1, pending)
    pending = nxt
  drain_writeback(_NCHUNK - 1, pending)


def _select32(x128, m):
  return jnp.where(
      m < 2,
      jnp.where(m == 0, x128[:, 0:EMB], x128[:, EMB:2 * EMB]),
      jnp.where(m == 2, x128[:, 2 * EMB:3 * EMB], x128[:, 3 * EMB:4 * EMB]))


def _mlp_body(item_ref, user_ref, dur_ref, mi_ref, mu_ref, md_ref,
              w1_ref, b1_ref, w2_ref, b2_ref, w3_ref, b3_ref, wo_ref, bo_ref,
              out_ref):
  f32 = jnp.float32
  xi = _select32(item_ref[...], mi_ref[...])
  xu = _select32(user_ref[...], mu_ref[...])
  xd = _select32(dur_ref[...], md_ref[...])
  h = jnp.dot(xi, w1_ref[0:EMB, :], preferred_element_type=f32)
  h += jnp.dot(xu, w1_ref[EMB:2 * EMB, :], preferred_element_type=f32)
  h += jnp.dot(xd, w1_ref[2 * EMB:3 * EMB, :], preferred_element_type=f32)
  h = jnp.maximum(h + b1_ref[...], 0.0)
  h = jnp.maximum(jnp.dot(h, w2_ref[...], preferred_element_type=f32) + b2_ref[...], 0.0)
  h = jnp.maximum(jnp.dot(h, w3_ref[...], preferred_element_type=f32) + b3_ref[...], 0.0)
  z = jnp.dot(h, wo_ref[...], preferred_element_type=f32) + bo_ref[...]
  out_ref[...] = 1.0 / (1.0 + jnp.exp(-z))


def kernel(user_id, item_id, duration, is_training, item_table, user_table,
           dur_table, W1, b1, W2, b2, W3, b3, Wo, bo):
  del is_training

  item_id = item_id.astype(jnp.int32)
  user_id = user_id.astype(jnp.int32)
  duration = duration.astype(jnp.int32)

  grp = lambda t: jnp.pad(t.reshape(-1), (0, (-t.size) % 128)).reshape(-1, 128)
  item_t128 = grp(item_table)
  user_t128 = grp(user_table)
  dur_t128 = dur_table.reshape(50, 128)

  gids = jnp.stack([
      (item_id // 4).reshape(BATCH // _CHUNK, _CHUNK),
      (user_id // 4).reshape(BATCH // _CHUNK, _CHUNK),
      (duration // 4).reshape(BATCH // _CHUNK, _CHUNK),
  ])

  mesh = plsc.VectorSubcoreMesh(core_axis_name="c", subcore_axis_name="s")
  wide = jax.ShapeDtypeStruct((BATCH, 128), jnp.float32)
  buf = pltpu.VMEM((_CHUNK, 128), jnp.float32)
  gather = functools.partial(
      pl.kernel,
      mesh=mesh,
      out_type=(wide, wide, wide),
      scratch_types=[
          pltpu.VMEM((3, _NCHUNK, _CHUNK), jnp.int32),
          buf, buf, buf, buf, buf, buf,
          pltpu.SemaphoreType.DMA,
          pltpu.SemaphoreType.DMA,
      ],
  )(_sc_gather_body)
  item_w, user_w, dur_w = gather(item_t128, user_t128, dur_t128, gids)

  bm = 2048
  grid = (BATCH // bm,)
  full = lambda shape: pl.BlockSpec(shape, lambda i: (0,) * len(shape))
  row = lambda w: pl.BlockSpec((bm, w), lambda i: (i, 0))
  out = pl.pallas_call(
      _mlp_body,
      grid=grid,
      in_specs=[
          row(128), row(128), row(128),
          row(1), row(1), row(1),
          full((3 * EMB, 128)),
          full((1, 128)),
          full((128, 64)),
          full((1, 64)),
          full((64, 32)),
          full((1, 32)),
          full((32, 2)),
          full((1, 2)),
      ],
      out_specs=pl.BlockSpec((bm, 2), lambda i: (i, 0)),
      out_shape=jax.ShapeDtypeStruct((BATCH, 2), jnp.float32),
  )(item_w, user_w, dur_w,
    (item_id % 4).reshape(BATCH, 1), (user_id % 4).reshape(BATCH, 1),
    (duration % 4).reshape(BATCH, 1),
    W1, b1.reshape(1, 128), W2, b2.reshape(1, 64), W3, b3.reshape(1, 32),
    Wo, bo.reshape(1, 2))
  return out
